# baseline (speedup 1.0000x reference)
import functools

import jax
import jax.numpy as jnp
from jax import lax
from jax.experimental import pallas as pl
from jax.experimental.pallas import tpu as pltpu
from jax.experimental.pallas import tpu_sc as plsc

N = 10000
D = 128
R = 8
E = 320000
NC = 2
NS = 16
NW = NC * NS
K0 = 80
NCH0 = (E // NW) // K0
K = 128
PCH = 160
NCHP = PCH * NS
CPAD = 81920
CSL = CPAD // NS
OWN = N // NC
DUM0 = 5120
AROWS = DUM0 + NS * 16
ASL = AROWS // NS


def _mesh():
    return plsc.VectorSubcoreMesh(core_axis_name="c", subcore_axis_name="s")


NCHD = (E // NS) // K0


def _deg_body(cid_hbm, out_hbm, cid_v, ones_v, zbuf_v, c0_v, cnt_sh, sem):
    c = lax.axis_index("c")
    s = lax.axis_index("s")

    @pl.when(c == 0)
    def _():
        def fill_ones(g, _):
            ones_v[pl.ds(g * 16, 16)] = jnp.ones((16,), jnp.float32)
            return 0

        lax.fori_loop(0, K0 // 16, fill_ones, 0)

        def fill_z(g, _):
            zbuf_v[pl.ds(g * 16, 16)] = jnp.zeros((16,), jnp.float32)
            return 0

        lax.fori_loop(0, CSL // 16, fill_z, 0)
        pltpu.sync_copy(zbuf_v, cnt_sh.at[pl.ds(s * CSL, CSL)])
        pltpu.sync_copy(cid_hbm.at[s], cid_v)
        plsc.subcore_barrier()
        prev = []
        for grp in range(10):
            cur = [
                pltpu.async_copy(ones_v, cnt_sh.at[cid_v.at[grp * 25 + j]],
                                 sem, add=True)
                for j in range(25)
            ]
            for dsc in prev:
                dsc.wait()
            prev = cur
        for dsc in prev:
            dsc.wait()
        plsc.subcore_barrier()
        prev = []
        for grp in range(10):
            cur = [
                pltpu.async_copy(cnt_sh.at[cid_v.at[grp * 25 + j]],
                                 c0_v.at[grp * 25 + j], sem)
                for j in range(25)
            ]
            for dsc in prev:
                dsc.wait()
            prev = cur
        for dsc in prev:
            dsc.wait()

        def nrm(i, _):
            r = i // 5
            g = i % 5
            c0 = c0_v[r, pl.ds(g * 16, 16)]
            c0_v[r, pl.ds(g * 16, 16)] = 1.0 / jnp.maximum(c0, 1.0)
            return 0

        lax.fori_loop(0, NCHD * 5, nrm, 0)
        pltpu.sync_copy(c0_v, out_hbm.at[s])


@functools.cache
def _deg_kernel():
    return pl.kernel(
        _deg_body,
        out_type=jax.ShapeDtypeStruct((NS, NCHD, K0), jnp.float32),
        mesh=_mesh(),
        scratch_types=[
            pltpu.VMEM((NCHD, K0), jnp.int32),
            pltpu.VMEM((K0,), jnp.float32),
            pltpu.VMEM((CSL,), jnp.float32),
            pltpu.VMEM((NCHD, K0), jnp.float32),
            pltpu.VMEM_SHARED((CPAD,), jnp.float32),
            pltpu.SemaphoreType.DMA,
        ],
    )


def _gss_body(xw_hbm, idx_hbm, out_hbm,
              idx_v, dstl_v, rows_v, norm_v,
              zbuf_v, agg_sh, isem0, isem1, isem2, isem3,
              rsem0, rsem1, rsem2, rsem3, ssem0, ssem1, ssem2, ssem3):
    c = lax.axis_index("c")
    s = lax.axis_index("s")
    isem = (isem0, isem1, isem2, isem3)
    rsem = (rsem0, rsem1, rsem2, rsem3)
    ssem = (ssem0, ssem1, ssem2, ssem3)
    base = s * PCH
    P = PCH

    def fill_z(i, _):
        zbuf_v[i // 8, pl.ds((i % 8) * 16, 16)] = jnp.zeros((16,), jnp.float32)
        return 0

    lax.fori_loop(0, 112 * 8, fill_z, 0)
    for q in range(3):
        pltpu.sync_copy(zbuf_v, agg_sh.at[pl.ds(s * ASL + q * 112, 112)])
    plsc.subcore_barrier()

    def idx_start(j, b):
        pltpu.async_copy(idx_hbm.at[base + j], idx_v.at[b], isem[b])

    def idx_wait(j, b):
        pltpu.make_async_copy(idx_hbm.at[base + j], idx_v.at[b], isem[b]).wait()

    def gather_start(b):
        pltpu.async_copy(xw_hbm.at[idx_v.at[b, 0]], rows_v.at[b], rsem[b])

    def gather_wait(b):
        pltpu.make_async_copy(xw_hbm.at[idx_v.at[b, 0]], rows_v.at[b], rsem[b]).wait()

    def scat_start(b):
        pltpu.async_copy(rows_v.at[b], agg_sh.at[dstl_v.at[b]], ssem[b], add=True)

    def scat_wait(b):
        pltpu.make_async_copy(rows_v.at[b], agg_sh.at[dstl_v.at[b]], ssem[b]).wait()

    def mkd(b):
        def f(g, _):
            loc = idx_v[b, 1, pl.ds(g * 16, 16)] - c * OWN
            ok = (loc >= 0) & (loc < OWN)
            dummy = DUM0 + s * 16 + lax.iota(jnp.int32, 16)
            dstl_v[b, pl.ds(g * 16, 16)] = jnp.where(ok, loc, dummy)
            norm_v[b, pl.ds(g * 16, 16)] = lax.bitcast_convert_type(
                idx_v[b, 2, pl.ds(g * 16, 16)], jnp.float32)
            return 0

        lax.fori_loop(0, K // 16, f, 0)

    def scale(b):
        def f(g, _):
            nv = norm_v[b, pl.ds(g * 16, 16)]
            for lane in range(16):
                nsp = jnp.full((16,), nv[lane], jnp.float32)
                row = g * 16 + lane
                for jj2 in range(8):
                    rows_v[b, row, pl.ds(jj2 * 16, 16)] = (
                        rows_v[b, row, pl.ds(jj2 * 16, 16)] * nsp)
            return 0

        lax.fori_loop(0, K // 16, f, 0)

    idx_start(0, 0)
    idx_wait(0, 0)
    gather_start(0)
    idx_start(1, 1)
    idx_wait(1, 1)
    gather_start(1)
    idx_start(2, 2)

    def quad(jt, _):
        for b in (0, 1, 2, 3):
            j = jt * 4 + b
            pb = (b + 2) % 4
            gather_wait(b)
            mkd(b)

            @pl.when(j <= P - 3)
            def _():
                idx_wait(j + 2, pb)

            @pl.when((j >= 2) & (j <= P - 3))
            def _():
                scat_wait(pb)

            @pl.when(j <= P - 3)
            def _():
                gather_start(pb)

            scale(b)
            scat_start(b)

            @pl.when(j <= P - 4)
            def _():
                idx_start(j + 3, (b + 3) % 4)
        return 0

    lax.fori_loop(0, P // 4, quad, 0)
    scat_wait(0)
    scat_wait(1)
    scat_wait(2)
    scat_wait(3)
    plsc.subcore_barrier()
    pltpu.sync_copy(agg_sh.at[pl.ds(s * ASL, ASL)],
                    out_hbm.at[c, pl.ds(s * ASL, ASL)])


@functools.cache
def _gss_kernel():
    return pl.kernel(
        _gss_body,
        out_type=jax.ShapeDtypeStruct((NC, AROWS, D), jnp.float32),
        mesh=_mesh(),
        scratch_types=[
            pltpu.VMEM((4, 3, K), jnp.int32),
            pltpu.VMEM((4, K), jnp.int32),
            pltpu.VMEM((4, K, D), jnp.float32),
            pltpu.VMEM((4, K), jnp.float32),
            pltpu.VMEM((112, D), jnp.float32),
            pltpu.VMEM_SHARED((AROWS, D), jnp.float32),
        ] + [pltpu.SemaphoreType.DMA] * 12,
    )


def _xw_body(x_ref, w_ref, o_ref):
    o_ref[...] = jnp.dot(x_ref[...], w_ref[0],
                         preferred_element_type=jnp.float32)[None]


def _xw(x, W):
    BN = 2000
    return pl.pallas_call(
        _xw_body,
        grid=(R, N // BN),
        in_specs=[
            pl.BlockSpec((BN, D), lambda r, n: (n, 0)),
            pl.BlockSpec((1, D, D), lambda r, n: (r, 0, 0)),
        ],
        out_specs=pl.BlockSpec((1, BN, D), lambda r, n: (r, n, 0)),
        out_shape=jax.ShapeDtypeStruct((R, N, D), jnp.float32),
    )(x, W)


def _cxw_body(p_ref, x_ref, wr_ref, b_ref, w2_ref, h_ref, o2_ref):
    h = jnp.maximum(
        p_ref[0]
        + jnp.dot(x_ref[...], wr_ref[...], preferred_element_type=jnp.float32)
        + b_ref[...], 0.0)
    h_ref[...] = h
    for r in range(R):
        o2_ref[r] = jnp.dot(h, w2_ref[r], preferred_element_type=jnp.float32)


def _cxw(p, x, wr, b, W2):
    BN = 1000
    nb = OWN // BN
    return pl.pallas_call(
        _cxw_body,
        grid=(N // BN,),
        in_specs=[
            pl.BlockSpec((1, BN, D), lambda n: (n // nb, n % nb, 0)),
            pl.BlockSpec((BN, D), lambda n: (n, 0)),
            pl.BlockSpec((D, D), lambda n: (0, 0)),
            pl.BlockSpec((1, D), lambda n: (0, 0)),
            pl.BlockSpec((R, D, D), lambda n: (0, 0, 0)),
        ],
        out_specs=[
            pl.BlockSpec((BN, D), lambda n: (n, 0)),
            pl.BlockSpec((R, BN, D), lambda n: (0, n, 0)),
        ],
        out_shape=[
            jax.ShapeDtypeStruct((N, D), jnp.float32),
            jax.ShapeDtypeStruct((R, N, D), jnp.float32),
        ],
    )(p, x, wr, b.reshape(1, D), W2)


def _combine_body(p_ref, x_ref, w_ref, b_ref, o_ref, *, relu):
    acc = (p_ref[0]
           + jnp.dot(x_ref[...], w_ref[...], preferred_element_type=jnp.float32)
           + b_ref[...])
    o_ref[...] = jnp.maximum(acc, 0.0) if relu else acc


def _combine(p, x, w, b, relu):
    BN = 1000
    nb = OWN // BN
    return pl.pallas_call(
        functools.partial(_combine_body, relu=relu),
        grid=(N // BN,),
        in_specs=[
            pl.BlockSpec((1, BN, D), lambda n: (n // nb, n % nb, 0)),
            pl.BlockSpec((BN, D), lambda n: (n, 0)),
            pl.BlockSpec((D, D), lambda n: (0, 0)),
            pl.BlockSpec((1, D), lambda n: (0, 0)),
        ],
        out_specs=pl.BlockSpec((BN, D), lambda n: (n, 0)),
        out_shape=jax.ShapeDtypeStruct((N, D), jnp.float32),
    )(p, x, w, b.reshape(1, D))


def kernel(x, edge_index, edge_type, W1, root1, b1, W2, root2, b2):
    src = edge_index[0].astype(jnp.int32)
    dst = edge_index[1].astype(jnp.int32)
    et = edge_type.astype(jnp.int32)
    cid = dst * R + et
    gidx = et * N + src
    cid16 = cid.reshape(NS, NCHD, K0)

    norm_e = _deg_kernel()(cid16).reshape(E)
    nbits = lax.bitcast_convert_type(norm_e, jnp.int32)
    npad = NCHP * K - E
    gpad = (jnp.arange(npad, dtype=jnp.int32) * 997) % (R * N)
    gidx_p = jnp.concatenate([gidx, gpad])
    dst_p = jnp.concatenate([dst, jnp.full((npad,), N, jnp.int32)])
    nrm_p = jnp.concatenate([nbits, jnp.zeros((npad,), jnp.int32)])
    idx3 = jnp.stack(
        [gidx_p.reshape(NCHP, K), dst_p.reshape(NCHP, K), nrm_p.reshape(NCHP, K)],
        axis=1)

    xw1 = _xw(x, W1).reshape(R * N, D)
    p1 = _gss_kernel()(xw1, idx3)
    h, xw2 = _cxw(p1, x, root1, b1, W2)
    p2 = _gss_kernel()(xw2.reshape(R * N, D), idx3)
    out = _combine(p2, h, root2, b2, relu=False)
    return out


# speedup vs baseline: 1.0264x; 1.0264x over previous
import functools

import jax
import jax.numpy as jnp
from jax import lax
from jax.experimental import pallas as pl
from jax.experimental.pallas import tpu as pltpu
from jax.experimental.pallas import tpu_sc as plsc

N = 10000
D = 128
R = 8
E = 320000
NC = 2
NS = 16
NW = NC * NS
K0 = 80
NCH0 = (E // NW) // K0
K = 128
PCH = 159
NCHP = PCH * NS
CPAD = 81920
CSL = CPAD // NS
OWN = N // NC
DUM0 = 5120
AROWS = DUM0 + NS * 16
ASL = AROWS // NS


def _mesh():
    return plsc.VectorSubcoreMesh(core_axis_name="c", subcore_axis_name="s")


NCHD = (E // NS) // K0


def _deg_body(cid_hbm, out_hbm, cid_v, ones_v, zbuf_v, c0_v, cnt_sh, sem):
    c = lax.axis_index("c")
    s = lax.axis_index("s")

    @pl.when(c == 0)
    def _():
        def fill_ones(g, _):
            ones_v[pl.ds(g * 16, 16)] = jnp.ones((16,), jnp.float32)
            return 0

        lax.fori_loop(0, K0 // 16, fill_ones, 0)

        def fill_z(g, _):
            zbuf_v[pl.ds(g * 16, 16)] = jnp.zeros((16,), jnp.float32)
            return 0

        lax.fori_loop(0, CSL // 16, fill_z, 0)
        pltpu.sync_copy(zbuf_v, cnt_sh.at[pl.ds(s * CSL, CSL)])
        pltpu.sync_copy(cid_hbm.at[s], cid_v)
        plsc.subcore_barrier()
        prev = []
        for grp in range(10):
            cur = [
                pltpu.async_copy(ones_v, cnt_sh.at[cid_v.at[grp * 25 + j]],
                                 sem, add=True)
                for j in range(25)
            ]
            for dsc in prev:
                dsc.wait()
            prev = cur
        for dsc in prev:
            dsc.wait()
        plsc.subcore_barrier()
        prev = []
        for grp in range(10):
            cur = [
                pltpu.async_copy(cnt_sh.at[cid_v.at[grp * 25 + j]],
                                 c0_v.at[grp * 25 + j], sem)
                for j in range(25)
            ]
            for dsc in prev:
                dsc.wait()
            prev = cur
        for dsc in prev:
            dsc.wait()

        def nrm(i, _):
            r = i // 5
            g = i % 5
            c0 = c0_v[r, pl.ds(g * 16, 16)]
            c0_v[r, pl.ds(g * 16, 16)] = 1.0 / jnp.maximum(c0, 1.0)
            return 0

        lax.fori_loop(0, NCHD * 5, nrm, 0)
        pltpu.sync_copy(c0_v, out_hbm.at[s])


@functools.cache
def _deg_kernel():
    return pl.kernel(
        _deg_body,
        out_type=jax.ShapeDtypeStruct((NS, NCHD, K0), jnp.float32),
        mesh=_mesh(),
        scratch_types=[
            pltpu.VMEM((NCHD, K0), jnp.int32),
            pltpu.VMEM((K0,), jnp.float32),
            pltpu.VMEM((CSL,), jnp.float32),
            pltpu.VMEM((NCHD, K0), jnp.float32),
            pltpu.VMEM_SHARED((CPAD,), jnp.float32),
            pltpu.SemaphoreType.DMA,
        ],
    )


def _gss_body(xw_hbm, idx_hbm, out_hbm,
              idx_v, dstl_v, rows_v, norm_v,
              zbuf_v, agg_sh, isem0, isem1, isem2, rsem0, rsem1, rsem2,
              ssem0, ssem1, ssem2):
    c = lax.axis_index("c")
    s = lax.axis_index("s")
    isem = (isem0, isem1, isem2)
    rsem = (rsem0, rsem1, rsem2)
    ssem = (ssem0, ssem1, ssem2)
    base = s * PCH
    P = PCH

    def fill_z(i, _):
        zbuf_v[i // 8, pl.ds((i % 8) * 16, 16)] = jnp.zeros((16,), jnp.float32)
        return 0

    lax.fori_loop(0, 112 * 8, fill_z, 0)
    for q in range(3):
        pltpu.sync_copy(zbuf_v, agg_sh.at[pl.ds(s * ASL + q * 112, 112)])
    plsc.subcore_barrier()

    def idx_start(j, b):
        pltpu.async_copy(idx_hbm.at[base + j], idx_v.at[b], isem[b])

    def idx_wait(j, b):
        pltpu.make_async_copy(idx_hbm.at[base + j], idx_v.at[b], isem[b]).wait()

    def gather_start(b):
        pltpu.async_copy(xw_hbm.at[idx_v.at[b, 0]], rows_v.at[b], rsem[b])

    def gather_wait(b):
        pltpu.make_async_copy(xw_hbm.at[idx_v.at[b, 0]], rows_v.at[b], rsem[b]).wait()

    def scat_start(b):
        pltpu.async_copy(rows_v.at[b], agg_sh.at[dstl_v.at[b]], ssem[b], add=True)

    def scat_wait(b):
        pltpu.make_async_copy(rows_v.at[b], agg_sh.at[dstl_v.at[b]], ssem[b]).wait()

    def mkd(b):
        def f(g, _):
            loc = idx_v[b, 1, pl.ds(g * 16, 16)] - c * OWN
            ok = (loc >= 0) & (loc < OWN)
            dummy = DUM0 + s * 16 + lax.iota(jnp.int32, 16)
            dstl_v[b, pl.ds(g * 16, 16)] = jnp.where(ok, loc, dummy)
            norm_v[b, pl.ds(g * 16, 16)] = lax.bitcast_convert_type(
                idx_v[b, 2, pl.ds(g * 16, 16)], jnp.float32)
            return 0

        lax.fori_loop(0, K // 16, f, 0)

    def scale(b):
        def f(g, _):
            nv = norm_v[b, pl.ds(g * 16, 16)]
            for lane in range(16):
                nsp = jnp.full((16,), nv[lane], jnp.float32)
                row = g * 16 + lane
                for jj2 in range(8):
                    rows_v[b, row, pl.ds(jj2 * 16, 16)] = (
                        rows_v[b, row, pl.ds(jj2 * 16, 16)] * nsp)
            return 0

        lax.fori_loop(0, K // 16, f, 0)

    idx_start(0, 0)
    idx_wait(0, 0)
    gather_start(0)
    idx_start(1, 1)
    idx_wait(1, 1)
    gather_start(1)
    idx_start(2, 2)

    def triple(jt, _):
        for b in (0, 1, 2):
            j = jt * 3 + b
            nb = (b + 1) % 3
            pb = (b + 2) % 3
            gather_wait(b)
            mkd(b)

            @pl.when(j <= P - 3)
            def _():
                idx_wait(j + 2, pb)

            scale(b)

            @pl.when((j >= 1) & (j <= P - 3))
            def _():
                scat_wait(pb)

            @pl.when(j <= P - 3)
            def _():
                gather_start(pb)

            scat_start(b)

            @pl.when(j <= P - 4)
            def _():
                idx_start(j + 3, b)
        return 0

    lax.fori_loop(0, P // 3, triple, 0)
    scat_wait(0)
    scat_wait(1)
    scat_wait(2)
    plsc.subcore_barrier()
    pltpu.sync_copy(agg_sh.at[pl.ds(s * ASL, ASL)],
                    out_hbm.at[c, pl.ds(s * ASL, ASL)])


@functools.cache
def _gss_kernel():
    return pl.kernel(
        _gss_body,
        out_type=jax.ShapeDtypeStruct((NC, AROWS, D), jnp.float32),
        mesh=_mesh(),
        scratch_types=[
            pltpu.VMEM((3, 3, K), jnp.int32),
            pltpu.VMEM((3, K), jnp.int32),
            pltpu.VMEM((3, K, D), jnp.float32),
            pltpu.VMEM((3, K), jnp.float32),
            pltpu.VMEM((112, D), jnp.float32),
            pltpu.VMEM_SHARED((AROWS, D), jnp.float32),
            pltpu.SemaphoreType.DMA,
            pltpu.SemaphoreType.DMA,
            pltpu.SemaphoreType.DMA,
            pltpu.SemaphoreType.DMA,
            pltpu.SemaphoreType.DMA,
            pltpu.SemaphoreType.DMA,
            pltpu.SemaphoreType.DMA,
            pltpu.SemaphoreType.DMA,
            pltpu.SemaphoreType.DMA,
        ],
    )


def _xw_body(x_ref, w_ref, o_ref):
    o_ref[...] = jnp.dot(x_ref[...], w_ref[0],
                         preferred_element_type=jnp.float32)[None]


def _xw(x, W):
    BN = 2000
    return pl.pallas_call(
        _xw_body,
        grid=(R, N // BN),
        in_specs=[
            pl.BlockSpec((BN, D), lambda r, n: (n, 0)),
            pl.BlockSpec((1, D, D), lambda r, n: (r, 0, 0)),
        ],
        out_specs=pl.BlockSpec((1, BN, D), lambda r, n: (r, n, 0)),
        out_shape=jax.ShapeDtypeStruct((R, N, D), jnp.float32),
    )(x, W)


def _cxw_body(p_ref, x_ref, wr_ref, b_ref, w2_ref, h_ref, o2_ref):
    h = jnp.maximum(
        p_ref[0]
        + jnp.dot(x_ref[...], wr_ref[...], preferred_element_type=jnp.float32)
        + b_ref[...], 0.0)
    h_ref[...] = h
    for r in range(R):
        o2_ref[r] = jnp.dot(h, w2_ref[r], preferred_element_type=jnp.float32)


def _cxw(p, x, wr, b, W2):
    BN = 1000
    nb = OWN // BN
    return pl.pallas_call(
        _cxw_body,
        grid=(N // BN,),
        in_specs=[
            pl.BlockSpec((1, BN, D), lambda n: (n // nb, n % nb, 0)),
            pl.BlockSpec((BN, D), lambda n: (n, 0)),
            pl.BlockSpec((D, D), lambda n: (0, 0)),
            pl.BlockSpec((1, D), lambda n: (0, 0)),
            pl.BlockSpec((R, D, D), lambda n: (0, 0, 0)),
        ],
        out_specs=[
            pl.BlockSpec((BN, D), lambda n: (n, 0)),
            pl.BlockSpec((R, BN, D), lambda n: (0, n, 0)),
        ],
        out_shape=[
            jax.ShapeDtypeStruct((N, D), jnp.float32),
            jax.ShapeDtypeStruct((R, N, D), jnp.float32),
        ],
    )(p, x, wr, b.reshape(1, D), W2)


def _combine_body(p_ref, x_ref, w_ref, b_ref, o_ref, *, relu):
    acc = (p_ref[0]
           + jnp.dot(x_ref[...], w_ref[...], preferred_element_type=jnp.float32)
           + b_ref[...])
    o_ref[...] = jnp.maximum(acc, 0.0) if relu else acc


def _combine(p, x, w, b, relu):
    BN = 1000
    nb = OWN // BN
    return pl.pallas_call(
        functools.partial(_combine_body, relu=relu),
        grid=(N // BN,),
        in_specs=[
            pl.BlockSpec((1, BN, D), lambda n: (n // nb, n % nb, 0)),
            pl.BlockSpec((BN, D), lambda n: (n, 0)),
            pl.BlockSpec((D, D), lambda n: (0, 0)),
            pl.BlockSpec((1, D), lambda n: (0, 0)),
        ],
        out_specs=pl.BlockSpec((BN, D), lambda n: (n, 0)),
        out_shape=jax.ShapeDtypeStruct((N, D), jnp.float32),
    )(p, x, w, b.reshape(1, D))


def kernel(x, edge_index, edge_type, W1, root1, b1, W2, root2, b2):
    src = edge_index[0].astype(jnp.int32)
    dst = edge_index[1].astype(jnp.int32)
    et = edge_type.astype(jnp.int32)
    cid = dst * R + et
    gidx = et * N + src
    cid16 = cid.reshape(NS, NCHD, K0)

    norm_e = _deg_kernel()(cid16).reshape(E)
    nbits = lax.bitcast_convert_type(norm_e, jnp.int32)
    npad = NCHP * K - E
    gpad = (jnp.arange(npad, dtype=jnp.int32) * 997) % (R * N)
    gidx_p = jnp.concatenate([gidx, gpad])
    dst_p = jnp.concatenate([dst, jnp.full((npad,), N, jnp.int32)])
    nrm_p = jnp.concatenate([nbits, jnp.zeros((npad,), jnp.int32)])
    idx3 = jnp.stack(
        [gidx_p.reshape(NCHP, K), dst_p.reshape(NCHP, K), nrm_p.reshape(NCHP, K)],
        axis=1)

    xw1 = _xw(x, W1).reshape(R * N, D)
    p1 = _gss_kernel()(xw1, idx3)
    h, xw2 = _cxw(p1, x, root1, b1, W2)
    p2 = _gss_kernel()(xw2.reshape(R * N, D), idx3)
    out = _combine(p2, h, root2, b2, relu=False)
    return out
